# baseline (speedup 1.0000x reference)
import functools

import jax
import jax.numpy as jnp
from jax import lax
from jax.experimental import pallas as pl
from jax.experimental.pallas import tpu as pltpu
from jax.experimental.pallas import tpu_sc as plsc

V = 10
VP = 16
D = 2048
B = 16384

S = 2048
BLK = 1024
NBLK = (B - S) // BLK

_info = plsc.get_sparse_core_info()
_NC = _info.num_cores
_NS = _info.num_subcores
NW = _NC * _NS
SPW = S // NW
C = 16
NCHUNK = SPW // C
TABW = V * D

_mesh = plsc.VectorSubcoreMesh(core_axis_name="c", subcore_axis_name="s")


@functools.partial(
    pl.kernel,
    mesh=_mesh,
    out_type=jax.ShapeDtypeStruct((B * D,), jnp.float32),
    scratch_types=[
        pltpu.VMEM((TABW,), jnp.float32),
        pltpu.VMEM((SPW,), jnp.int32),
        pltpu.VMEM((16,), jnp.float32),
        pltpu.SemaphoreType.DMA,
    ],
)
def _sc_lookup(tab_hbm, idx_hbm, scl_hbm, out_hbm, tab_v, idx_v, scl_v, wsem):
    wid = lax.axis_index("s") * _NC + lax.axis_index("c")
    base = wid * SPW * D

    pltpu.sync_copy(tab_hbm, tab_v)
    pltpu.sync_copy(idx_hbm.at[wid], idx_v)
    pltpu.sync_copy(scl_hbm, scl_v)
    s = scl_v[...]

    @plsc.parallel_loop(0, TABW, step=16, unroll=8)
    def _(j):
        sl = pl.ds(j, 16)
        tab_v[sl] = tab_v[sl] * s

    def k_body(k, carry):
        ids16 = idx_v[pl.ds(k * C, C)]
        for r in range(C):
            pltpu.async_copy(
                tab_v.at[pl.ds(ids16[r] * D, D)],
                out_hbm.at[pl.ds(base + (k * C + r) * D, D)],
                wsem,
            )
        return carry

    lax.fori_loop(0, NCHUNK, k_body, 0)

    def drain(j, c):
        pltpu.make_async_copy(
            tab_v.at[pl.ds(0, D)], out_hbm.at[pl.ds(0, D)], wsem
        ).wait()
        return c

    lax.fori_loop(0, SPW, drain, 0)


def _tc_body(s_ref, ids_ref, tab_ref, prev_ref, o_ref):
    del prev_ref
    ids = ids_ref[0, 0, :]
    onehot = jnp.where(
        ids[:, None] == lax.broadcasted_iota(jnp.int32, (BLK, VP), 1),
        s_ref[0], 0.0,
    )
    o_ref[...] = jnp.dot(onehot, tab_ref[...],
                         preferred_element_type=jnp.float32)


def _tc_lookup(ids3d, tab_p, scale1, out_partial):
    return pl.pallas_call(
        _tc_body,
        grid=(NBLK,),
        in_specs=[
            pl.BlockSpec(memory_space=pltpu.SMEM),
            pl.BlockSpec((1, 1, BLK), lambda i: (i, 0, 0)),
            pl.BlockSpec((VP, D), lambda i: (0, 0)),
            pl.BlockSpec(memory_space=pltpu.MemorySpace.HBM),
        ],
        out_specs=pl.BlockSpec((BLK, D), lambda i: (i + S // BLK, 0)),
        out_shape=jax.ShapeDtypeStruct((B, D), jnp.float32),
        input_output_aliases={3: 0},
    )(scale1, ids3d, tab_p, out_partial)


def kernel(inscription_ids, embedding, scale):
    ids = inscription_ids.astype(jnp.int32)
    out_sc = _sc_lookup(
        embedding.reshape(-1),
        ids[:S].reshape(NW, SPW),
        jnp.broadcast_to(scale, (16,)),
    )
    tab_p = jnp.pad(embedding, ((0, VP - V), (0, 0)))
    return _tc_lookup(
        ids[S:].reshape(NBLK, 1, BLK), tab_p, jnp.reshape(scale, (1,)),
        out_sc.reshape(B, D),
    )


# speedup vs baseline: 2.7731x; 2.7731x over previous
import functools

import jax
import jax.numpy as jnp
from jax import lax
from jax.experimental import pallas as pl
from jax.experimental.pallas import tpu as pltpu
from jax.experimental.pallas import tpu_sc as plsc

V = 10
VP = 16
D = 2048
B = 16384

S = 2048
BLK = 1024
NBLK = (B - S) // BLK

_info = plsc.get_sparse_core_info()
_NC = _info.num_cores
_NS = _info.num_subcores
NW = _NC * _NS
SPW = S // NW
C = 16
NCHUNK = SPW // C
TABW = V * D

_mesh = plsc.VectorSubcoreMesh(core_axis_name="c", subcore_axis_name="s")


@functools.partial(
    pl.kernel,
    mesh=_mesh,
    out_type=jax.ShapeDtypeStruct((B, D), jnp.float32),
    scratch_types=[
        pltpu.VMEM((V, D), jnp.float32),
        pltpu.VMEM((SPW,), jnp.int32),
        pltpu.VMEM((16,), jnp.float32),
        pltpu.SemaphoreType.DMA,
    ],
)
def _sc_lookup(tab_hbm, idx_hbm, scl_hbm, out_hbm, tab_v, idx_v, scl_v, wsem):
    wid = lax.axis_index("s") * _NC + lax.axis_index("c")
    base = wid * SPW

    pltpu.sync_copy(tab_hbm, tab_v)
    pltpu.sync_copy(idx_hbm.at[wid], idx_v)
    pltpu.sync_copy(scl_hbm, scl_v)
    s = scl_v[...]

    for v in range(V):
        @plsc.parallel_loop(0, D, step=16, unroll=8)
        def _(j):
            sl = pl.ds(j, 16)
            tab_v[v, sl] = tab_v[v, sl] * s

    tab2d = tab_v

    def k_body(k, carry):
        ids16 = idx_v[pl.ds(k * C, C)]
        for r in range(C):
            pltpu.async_copy(
                tab2d.at[pl.ds(ids16[r], 1)],
                out_hbm.at[pl.ds(base + k * C + r, 1)],
                wsem,
            )
        return carry

    lax.fori_loop(0, NCHUNK, k_body, 0)

    def drain(j, c):
        pltpu.make_async_copy(
            tab2d.at[pl.ds(0, 1)], out_hbm.at[pl.ds(0, 1)], wsem
        ).wait()
        return c

    lax.fori_loop(0, SPW, drain, 0)


def _tc_body(s_ref, ids_ref, tab_ref, prev_ref, o_ref):
    del prev_ref
    ids = ids_ref[0, 0, :]
    onehot = jnp.where(
        ids[:, None] == lax.broadcasted_iota(jnp.int32, (BLK, VP), 1),
        s_ref[0], 0.0,
    )
    o_ref[...] = jnp.dot(onehot, tab_ref[...],
                         preferred_element_type=jnp.float32)


def _tc_lookup(ids3d, tab_p, scale1, out_partial):
    return pl.pallas_call(
        _tc_body,
        grid=(NBLK,),
        in_specs=[
            pl.BlockSpec(memory_space=pltpu.SMEM),
            pl.BlockSpec((1, 1, BLK), lambda i: (i, 0, 0)),
            pl.BlockSpec((VP, D), lambda i: (0, 0)),
            pl.BlockSpec(memory_space=pltpu.MemorySpace.HBM),
        ],
        out_specs=pl.BlockSpec((BLK, D), lambda i: (i + S // BLK, 0)),
        out_shape=jax.ShapeDtypeStruct((B, D), jnp.float32),
        input_output_aliases={3: 0},
    )(scale1, ids3d, tab_p, out_partial)


def kernel(inscription_ids, embedding, scale):
    ids = inscription_ids.astype(jnp.int32)
    out_sc = _sc_lookup(
        embedding,
        ids[:S].reshape(NW, SPW),
        jnp.broadcast_to(scale, (16,)),
    )
    tab_p = jnp.pad(embedding, ((0, VP - V), (0, 0)))
    return _tc_lookup(
        ids[S:].reshape(NBLK, 1, BLK), tab_p, jnp.reshape(scale, (1,)),
        out_sc,
    )


# speedup vs baseline: 2.8128x; 1.0143x over previous
import functools

import jax
import jax.numpy as jnp
from jax import lax
from jax.experimental import pallas as pl
from jax.experimental.pallas import tpu as pltpu
from jax.experimental.pallas import tpu_sc as plsc

V = 10
VP = 16
D = 2048
B = 16384

S = 1024
BLK = 1024
NBLK = (B - S) // BLK

_info = plsc.get_sparse_core_info()
_NC = _info.num_cores
_NS = _info.num_subcores
NW = _NC * _NS
SPW = S // NW
C = 16
NCHUNK = SPW // C
TABW = V * D

_mesh = plsc.VectorSubcoreMesh(core_axis_name="c", subcore_axis_name="s")


@functools.partial(
    pl.kernel,
    mesh=_mesh,
    out_type=jax.ShapeDtypeStruct((B, D), jnp.float32),
    scratch_types=[
        pltpu.VMEM((V, D), jnp.float32),
        pltpu.VMEM((SPW,), jnp.int32),
        pltpu.VMEM((16,), jnp.float32),
        pltpu.SemaphoreType.DMA,
    ],
)
def _sc_lookup(tab_hbm, idx_hbm, scl_hbm, out_hbm, tab_v, idx_v, scl_v, wsem):
    wid = lax.axis_index("s") * _NC + lax.axis_index("c")
    base = wid * SPW

    pltpu.sync_copy(tab_hbm, tab_v)
    pltpu.sync_copy(idx_hbm.at[wid], idx_v)
    pltpu.sync_copy(scl_hbm, scl_v)
    s = scl_v[...]

    for v in range(V):
        @plsc.parallel_loop(0, D, step=16, unroll=8)
        def _(j):
            sl = pl.ds(j, 16)
            tab_v[v, sl] = tab_v[v, sl] * s

    tab2d = tab_v

    def k_body(k, carry):
        ids16 = idx_v[pl.ds(k * C, C)]
        for r in range(C):
            pltpu.async_copy(
                tab2d.at[pl.ds(ids16[r], 1)],
                out_hbm.at[pl.ds(base + k * C + r, 1)],
                wsem,
            )
        return carry

    lax.fori_loop(0, NCHUNK, k_body, 0)

    def drain(j, c):
        pltpu.make_async_copy(
            tab2d.at[pl.ds(0, 1)], out_hbm.at[pl.ds(0, 1)], wsem
        ).wait()
        return c

    lax.fori_loop(0, SPW, drain, 0)


def _tc_body(s_ref, ids_ref, tab_ref, prev_ref, o_ref):
    del prev_ref
    ids = ids_ref[0, 0, :]
    onehot = jnp.where(
        ids[:, None] == lax.broadcasted_iota(jnp.int32, (BLK, VP), 1),
        s_ref[0], 0.0,
    )
    o_ref[...] = jnp.dot(onehot, tab_ref[...],
                         preferred_element_type=jnp.float32)


def _tc_lookup(ids3d, tab_p, scale1, out_partial):
    return pl.pallas_call(
        _tc_body,
        grid=(NBLK,),
        in_specs=[
            pl.BlockSpec(memory_space=pltpu.SMEM),
            pl.BlockSpec((1, 1, BLK), lambda i: (i, 0, 0)),
            pl.BlockSpec((VP, D), lambda i: (0, 0)),
            pl.BlockSpec(memory_space=pltpu.MemorySpace.HBM),
        ],
        out_specs=pl.BlockSpec((BLK, D), lambda i: (i + S // BLK, 0)),
        out_shape=jax.ShapeDtypeStruct((B, D), jnp.float32),
        input_output_aliases={3: 0},
    )(scale1, ids3d, tab_p, out_partial)


def kernel(inscription_ids, embedding, scale):
    ids = inscription_ids.astype(jnp.int32)
    out_sc = _sc_lookup(
        embedding,
        ids[:S].reshape(NW, SPW),
        jnp.broadcast_to(scale, (16,)),
    )
    tab_p = jnp.pad(embedding, ((0, VP - V), (0, 0)))
    return _tc_lookup(
        ids[S:].reshape(NBLK, 1, BLK), tab_p, jnp.reshape(scale, (1,)),
        out_sc,
    )
